# baseline (speedup 1.0000x reference)
import functools

import jax
import jax.numpy as jnp
from jax import lax
from jax.experimental import pallas as pl
from jax.experimental.pallas import tpu as pltpu
from jax.experimental.pallas import tpu_sc as plsc

_DIM = 1000
_N_ROWS = 1024 * 26
_NC = 2
_NS = 16
_NW = _NC * _NS
_ROWS_PER_W = _N_ROWS // _NW
_C = 32
_GROUPS = _C // 16
_N_CHUNKS = _ROWS_PER_W // _C


@functools.partial(
    pl.kernel,
    out_type=jax.ShapeDtypeStruct((_N_ROWS, _DIM), jnp.float32),
    mesh=plsc.VectorSubcoreMesh(core_axis_name="c", subcore_axis_name="s"),
    compiler_params=pltpu.CompilerParams(
        use_tc_tiling_on_sc=False, needs_layout_passes=False
    ),
    scratch_types=[
        pltpu.VMEM((_ROWS_PER_W,), jnp.int32),
        pltpu.VMEM((_C, _DIM), jnp.float32),
        pltpu.VMEM((_C, _DIM), jnp.float32),
        pltpu.SemaphoreType.DMA,
        pltpu.SemaphoreType.DMA,
    ],
)
def _one_hot_sc(idx_hbm, zeros_hbm, out_hbm, idx_v, buf_a, buf_b, sem_a, sem_b):
    wid = lax.axis_index("s") * _NC + lax.axis_index("c")
    base = wid * _ROWS_PER_W

    pltpu.sync_copy(idx_hbm.at[pl.ds(base, _ROWS_PER_W)], idx_v)
    pltpu.sync_copy(zeros_hbm, buf_a)
    pltpu.sync_copy(zeros_hbm, buf_b)

    bufs = (buf_a, buf_b)
    sems = (sem_a, sem_b)
    lane = lax.iota(jnp.int32, 16)
    ones = jnp.ones((16,), jnp.float32)
    zeros = jnp.zeros((16,), jnp.float32)
    pending = [None, None]

    for k in range(_N_CHUNKS):
        b = k % 2
        buf = bufs[b]
        if pending[b] is not None:
            pending[b].wait()
            for g in range(_GROUPS):
                rows = lane + (g * 16)
                cols = idx_v[pl.ds((k - 2) * _C + g * 16, 16)]
                plsc.store_scatter(buf, [rows, cols], zeros)
        for g in range(_GROUPS):
            rows = lane + (g * 16)
            cols = idx_v[pl.ds(k * _C + g * 16, 16)]
            plsc.store_scatter(buf, [rows, cols], ones)
        pending[b] = pltpu.async_copy(
            buf, out_hbm.at[pl.ds(base + k * _C, _C)], sems[b]
        )
    pending[0].wait()
    pending[1].wait()


def kernel(tensor):
    idx = tensor.reshape(_N_ROWS).astype(jnp.int32)
    zeros = jnp.zeros((_C, _DIM), jnp.float32)
    out = _one_hot_sc(idx, zeros)
    return out.reshape(tensor.shape + (_DIM,))
